# baseline (speedup 1.0000x reference)
import functools

import jax
import jax.numpy as jnp
from jax import lax
from jax.experimental import pallas as pl
from jax.experimental.pallas import tpu as pltpu
from jax.experimental.pallas import tpu_sc as plsc

_N = 10000
_NP = 10240
_E = 320000
_K = 128
_NT = 32
_TCH = 2560
_EP = _TCH * _K
_NCH0 = 160
_NCH1 = 0
_DUMMY = _N
_RPT = _NP // 16
_NB = 8
_LA = 4
_EPT_H = _EP // _NT


def _sc_mesh():
    return plsc.VectorSubcoreMesh(core_axis_name="c", subcore_axis_name="s")


@functools.partial(
    pl.kernel,
    mesh=_sc_mesh(),
    compiler_params=pltpu.CompilerParams(needs_layout_passes=False),
    out_type=jax.ShapeDtypeStruct((_NT, _NP), jnp.int32),
    scratch_types=[
        pltpu.VMEM((_NP,), jnp.int32),
        pltpu.VMEM((_EPT_H,), jnp.int32),
    ],
)
def _sc_hist(dst_hbm, out_hbm, hist_v, idx_v):
    c = lax.axis_index("c")
    s = lax.axis_index("s")
    wid = c * 16 + s

    def zero(j, carry):
        hist_v[pl.ds(j * 16, 16)] = jnp.zeros((16,), jnp.int32)
        return carry

    lax.fori_loop(0, _NP // 16, zero, 0)
    pltpu.sync_copy(dst_hbm.at[pl.ds(wid * _EPT_H, _EPT_H)], idx_v)

    ones = jnp.ones((16,), jnp.int32)

    def body(j, carry):
        idx = idx_v[pl.ds(j * 16, 16)]
        plsc.addupdate_scatter(hist_v, [idx], ones)
        return carry

    lax.fori_loop(0, _EPT_H // 16, body, 0)
    pltpu.sync_copy(hist_v, out_hbm.at[wid])


@functools.partial(
    pl.kernel,
    mesh=_sc_mesh(),
    compiler_params=pltpu.CompilerParams(
        needs_layout_passes=False, use_tc_tiling_on_sc=False),
    out_type=jax.ShapeDtypeStruct((2, _NP, 16), jnp.float32),
    scratch_types=[
        pltpu.VMEM((_NCH0, _K), jnp.int32),
        pltpu.VMEM((_NCH0, _K), jnp.int32),
        pltpu.VMEM((_NB, _K, 16), jnp.float32),
        pltpu.VMEM_SHARED((_NP, 16), jnp.float32),
        pltpu.SemaphoreType.DMA((_NB,)),
        pltpu.SemaphoreType.DMA((_NB,)),
    ],
)
def _sc_agg(src_hbm, dst_hbm, hs_hbm, zeros_hbm, out_hbm,
            idx_s, idx_d, rows, acc_sh, gsem, ssem):
    c = lax.axis_index("c")
    s = lax.axis_index("s")

    r0 = s * _RPT

    @pl.when(c == 0)
    def _init_hs():
        pltpu.sync_copy(src_hbm.at[pl.ds(s * _NCH0, _NCH0)], idx_s)
        pltpu.sync_copy(dst_hbm.at[pl.ds(s * _NCH0, _NCH0)], idx_d)
        pltpu.sync_copy(hs_hbm.at[pl.ds(r0, _RPT)], acc_sh.at[pl.ds(r0, _RPT)])

    @pl.when(c == 1)
    def _init_zero():
        pltpu.sync_copy(zeros_hbm, acc_sh.at[pl.ds(r0, _RPT)])

    plsc.subcore_barrier()

    def gat(cc, b):
        pltpu.async_copy(hs_hbm.at[idx_s.at[cc]], rows.at[b], gsem.at[b])

    def gwait(b):
        pltpu.make_async_copy(
            hs_hbm.at[idx_s.at[0]], rows.at[b], gsem.at[b]).wait()

    def scat(cc, b):
        pltpu.async_copy(rows.at[b], acc_sh.at[idx_d.at[cc]], ssem.at[b],
                         add=True)

    def swait(b):
        pltpu.make_async_copy(
            rows.at[b], acc_sh.at[idx_d.at[0]], ssem.at[b]).wait()

    def pipeline(nch):
        for b in range(_LA):
            gat(b, b)
        for j in range(_LA):
            gwait(j)
            scat(j, j)
            gat(j + _LA, j + _LA)

        def rnd(r, carry):
            base = _LA + r * _NB
            for j in range(_NB):
                b = (_LA + j) % _NB
                pb = (b + _LA) % _NB
                gwait(b)
                scat(base + j, b)
                swait(pb)
                gat(base + j + _LA, pb)
            return carry

        lax.fori_loop(0, (nch - 2 * _LA) // _NB, rnd, 0)
        for j in range(_LA):
            b = (_LA + j) % _NB
            gwait(b)
            scat(nch - _LA + j, b)
            swait((b + _LA) % _NB)
        for j in range(_LA):
            swait((_LA + j) % _NB)

    @pl.when(c == 0)
    def _run0():
        pipeline(_NCH0)

    plsc.subcore_barrier()
    pltpu.sync_copy(acc_sh.at[pl.ds(r0, _RPT)],
                    out_hbm.at[c, pl.ds(r0, _RPT)])


def _tc1_body(hist_ref, x_ref, w1_ref, hs1_ref, dinv_ref):
    hist_f = hist_ref[...].astype(jnp.float32)
    ones = jnp.ones((_NT, 1), jnp.float32)
    deg = lax.dot_general(hist_f, ones, (((0,), (0,)), ((), ())),
                          preferred_element_type=jnp.float32)
    dinv = lax.rsqrt(deg + 1.0)
    dinv_ref[...] = dinv
    h1 = jnp.dot(x_ref[...], w1_ref[...], preferred_element_type=jnp.float32)
    hs1_ref[...] = jnp.concatenate(
        [h1 * dinv[:_N], jnp.zeros((_NP - _N, 16), jnp.float32)], axis=0)


_tc1 = pl.pallas_call(
    _tc1_body,
    out_shape=[
        jax.ShapeDtypeStruct((_NP, 16), jnp.float32),
        jax.ShapeDtypeStruct((_NP, 1), jnp.float32),
    ],
)


def _tc2_body(a0_ref, a1_ref, dinv_ref, b1_ref, w2_ref, hs2_ref):
    acc = a0_ref[...] + a1_ref[...]
    pre = acc * dinv_ref[...] + b1_ref[...]
    out1 = jnp.maximum(pre, 0.0)
    rows = lax.broadcasted_iota(jnp.int32, (_NP, 16), 0)
    out1 = jnp.where(rows < _N, out1, 0.0)
    h2 = jnp.dot(out1, w2_ref[...], preferred_element_type=jnp.float32)
    hs2_ref[...] = h2 * dinv_ref[...]


_tc2 = pl.pallas_call(
    _tc2_body,
    out_shape=jax.ShapeDtypeStruct((_NP, 16), jnp.float32),
)


def _tc3_body(a0_ref, a1_ref, dinv_ref, b2_ref, out_ref):
    logits = (a0_ref[...] + a1_ref[...]) * dinv_ref[...] + b2_ref[...]
    m = jnp.max(logits, axis=1, keepdims=True)
    lse = jnp.log(jnp.sum(jnp.exp(logits - m), axis=1, keepdims=True)) + m
    out_ref[...] = (logits - lse)[:_N]


_tc3 = pl.pallas_call(
    _tc3_body,
    out_shape=jax.ShapeDtypeStruct((_N, 16), jnp.float32),
)


def kernel(x, edge_index, W1, b1, W2, b2):
    pad = jnp.full((_EP - _E,), _DUMMY, jnp.int32)
    src2 = jnp.concatenate([edge_index[0], pad]).reshape(_TCH, _K)
    dst2 = jnp.concatenate([edge_index[1], pad]).reshape(_TCH, _K)
    dst_flat = dst2.reshape(_EP)
    zeros_rows = jnp.zeros((_RPT, 16), jnp.float32)

    hist = _sc_hist(dst_flat)
    hs1, dinv = _tc1(hist, x, W1)

    acc1 = _sc_agg(src2, dst2, hs1, zeros_rows)
    hs2 = _tc2(acc1[0], acc1[1], dinv, b1.reshape(1, 16), W2)

    acc2 = _sc_agg(src2, dst2, hs2, zeros_rows)
    return _tc3(acc2[0], acc2[1], dinv, b2.reshape(1, 16))


# speedup vs baseline: 1.2196x; 1.2196x over previous
import functools

import jax
import jax.numpy as jnp
from jax import lax
from jax.experimental import pallas as pl
from jax.experimental.pallas import tpu as pltpu
from jax.experimental.pallas import tpu_sc as plsc

_N = 10000
_NP = 10240
_E = 320000
_K = 128
_NT = 32
_TCH = 2560
_EP = _TCH * _K
_NCH0 = 120
_NCH1 = 40
_DUMMY = _N
_RPT = _NP // 16
_NB = 8
_LA = 4
_EPT_H = _EP // _NT


def _sc_mesh():
    return plsc.VectorSubcoreMesh(core_axis_name="c", subcore_axis_name="s")


@functools.partial(
    pl.kernel,
    mesh=_sc_mesh(),
    compiler_params=pltpu.CompilerParams(needs_layout_passes=False),
    out_type=jax.ShapeDtypeStruct((_NT, _NP), jnp.int32),
    scratch_types=[
        pltpu.VMEM((_NP,), jnp.int32),
        pltpu.VMEM((_EPT_H,), jnp.int32),
    ],
)
def _sc_hist(dst_hbm, out_hbm, hist_v, idx_v):
    c = lax.axis_index("c")
    s = lax.axis_index("s")
    wid = c * 16 + s

    def zero(j, carry):
        hist_v[pl.ds(j * 16, 16)] = jnp.zeros((16,), jnp.int32)
        return carry

    lax.fori_loop(0, _NP // 16, zero, 0)
    pltpu.sync_copy(dst_hbm.at[pl.ds(wid * _EPT_H, _EPT_H)], idx_v)

    ones = jnp.ones((16,), jnp.int32)

    def body(j, carry):
        idx = idx_v[pl.ds(j * 16, 16)]
        plsc.addupdate_scatter(hist_v, [idx], ones)
        return carry

    lax.fori_loop(0, _EPT_H // 16, body, 0)
    pltpu.sync_copy(hist_v, out_hbm.at[wid])


@functools.partial(
    pl.kernel,
    mesh=_sc_mesh(),
    compiler_params=pltpu.CompilerParams(
        needs_layout_passes=False, use_tc_tiling_on_sc=False),
    out_type=jax.ShapeDtypeStruct((2, _NP, 16), jnp.float32),
    scratch_types=[
        pltpu.VMEM((_NCH0, _K), jnp.int32),
        pltpu.VMEM((_NCH0, _K), jnp.int32),
        pltpu.VMEM((_NB, _K, 16), jnp.float32),
        pltpu.VMEM_SHARED((_NP, 16), jnp.float32),
        pltpu.SemaphoreType.DMA((_NB,)),
        pltpu.SemaphoreType.DMA((_NB,)),
    ],
)
def _sc_agg(src_hbm, dst_hbm, hs_hbm, zeros_hbm, out_hbm,
            idx_s, idx_d, rows, acc_sh, gsem, ssem):
    c = lax.axis_index("c")
    s = lax.axis_index("s")

    r0 = s * _RPT

    @pl.when(c == 0)
    def _init_hs():
        pltpu.sync_copy(src_hbm.at[pl.ds(s * _NCH0, _NCH0)], idx_s)
        pltpu.sync_copy(dst_hbm.at[pl.ds(s * _NCH0, _NCH0)], idx_d)
        pltpu.sync_copy(hs_hbm.at[pl.ds(r0, _RPT)], acc_sh.at[pl.ds(r0, _RPT)])

    @pl.when(c == 1)
    def _init_zero():
        base = _NT // 2 * _NCH0 + s * _NCH1
        pltpu.sync_copy(src_hbm.at[pl.ds(base, _NCH1)],
                        idx_s.at[pl.ds(0, _NCH1)])
        pltpu.sync_copy(dst_hbm.at[pl.ds(base, _NCH1)],
                        idx_d.at[pl.ds(0, _NCH1)])
        pltpu.sync_copy(zeros_hbm, acc_sh.at[pl.ds(r0, _RPT)])

    plsc.subcore_barrier()

    def gat(cc, b):
        pltpu.async_copy(hs_hbm.at[idx_s.at[cc]], rows.at[b], gsem.at[b])

    def gwait(b):
        pltpu.make_async_copy(
            hs_hbm.at[idx_s.at[0]], rows.at[b], gsem.at[b]).wait()

    def scat(cc, b):
        pltpu.async_copy(rows.at[b], acc_sh.at[idx_d.at[cc]], ssem.at[b],
                         add=True)

    def swait(b):
        pltpu.make_async_copy(
            rows.at[b], acc_sh.at[idx_d.at[0]], ssem.at[b]).wait()

    def pipeline(nch):
        for b in range(_LA):
            gat(b, b)
        for j in range(_LA):
            gwait(j)
            scat(j, j)
            gat(j + _LA, j + _LA)

        def rnd(r, carry):
            base = _LA + r * _NB
            for j in range(_NB):
                b = (_LA + j) % _NB
                pb = (b + _LA) % _NB
                gwait(b)
                scat(base + j, b)
                swait(pb)
                gat(base + j + _LA, pb)
            return carry

        lax.fori_loop(0, (nch - 2 * _LA) // _NB, rnd, 0)
        for j in range(_LA):
            b = (_LA + j) % _NB
            gwait(b)
            scat(nch - _LA + j, b)
            swait((b + _LA) % _NB)
        for j in range(_LA):
            swait((_LA + j) % _NB)

    @pl.when(c == 0)
    def _run0():
        pipeline(_NCH0)

    @pl.when(c == 1)
    def _run1():
        pipeline(_NCH1)

    plsc.subcore_barrier()
    pltpu.sync_copy(acc_sh.at[pl.ds(r0, _RPT)],
                    out_hbm.at[c, pl.ds(r0, _RPT)])


def _tc1_body(hist_ref, x_ref, w1_ref, hs1_ref, dinv_ref):
    hist_f = hist_ref[...].astype(jnp.float32)
    ones = jnp.ones((_NT, 1), jnp.float32)
    deg = lax.dot_general(hist_f, ones, (((0,), (0,)), ((), ())),
                          preferred_element_type=jnp.float32)
    dinv = lax.rsqrt(deg + 1.0)
    dinv_ref[...] = dinv
    h1 = jnp.dot(x_ref[...], w1_ref[...], preferred_element_type=jnp.float32)
    hs1_ref[...] = jnp.concatenate(
        [h1 * dinv[:_N], jnp.zeros((_NP - _N, 16), jnp.float32)], axis=0)


_tc1 = pl.pallas_call(
    _tc1_body,
    out_shape=[
        jax.ShapeDtypeStruct((_NP, 16), jnp.float32),
        jax.ShapeDtypeStruct((_NP, 1), jnp.float32),
    ],
)


def _tc2_body(a_ref, dinv_ref, b1_ref, w2_ref, hs2_ref):
    acc = a_ref[0:_NP] + a_ref[_NP:2 * _NP]
    pre = acc * dinv_ref[...] + b1_ref[...]
    out1 = jnp.maximum(pre, 0.0)
    rows = lax.broadcasted_iota(jnp.int32, (_NP, 16), 0)
    out1 = jnp.where(rows < _N, out1, 0.0)
    h2 = jnp.dot(out1, w2_ref[...], preferred_element_type=jnp.float32)
    hs2_ref[...] = h2 * dinv_ref[...]


_tc2 = pl.pallas_call(
    _tc2_body,
    out_shape=jax.ShapeDtypeStruct((_NP, 16), jnp.float32),
)


def _tc3_body(a_ref, dinv_ref, b2_ref, out_ref):
    logits = (a_ref[0:_NP] + a_ref[_NP:2 * _NP]) * dinv_ref[...] \
        + b2_ref[...]
    m = jnp.max(logits, axis=1, keepdims=True)
    lse = jnp.log(jnp.sum(jnp.exp(logits - m), axis=1, keepdims=True)) + m
    out_ref[...] = (logits - lse)[:_N]


_tc3 = pl.pallas_call(
    _tc3_body,
    out_shape=jax.ShapeDtypeStruct((_N, 16), jnp.float32),
)


def kernel(x, edge_index, W1, b1, W2, b2):
    pad = jnp.full((_EP - _E,), _DUMMY, jnp.int32)
    src2 = jnp.concatenate([edge_index[0], pad]).reshape(_TCH, _K)
    dst2 = jnp.concatenate([edge_index[1], pad]).reshape(_TCH, _K)
    dst_flat = dst2.reshape(_EP)
    zeros_rows = jnp.zeros((_RPT, 16), jnp.float32)

    hist = _sc_hist(dst_flat)
    hs1, dinv = _tc1(hist, x, W1)

    acc1 = _sc_agg(src2, dst2, hs1, zeros_rows)
    hs2 = _tc2(acc1.reshape(2 * _NP, 16), dinv, b1.reshape(1, 16), W2)

    acc2 = _sc_agg(src2, dst2, hs2, zeros_rows)
    return _tc3(acc2.reshape(2 * _NP, 16), dinv, b2.reshape(1, 16))
